# baseline (speedup 1.0000x reference)
import functools

import jax
import jax.numpy as jnp
from jax import lax
from jax.experimental import pallas as pl
from jax.experimental.pallas import tpu as pltpu
from jax.experimental.pallas import tpu_sc as plsc

NC = 2
NS = 16
L = 16
NW = NC * NS
CHUNK = 64


def _sc_partials(o2d, b2d, n_rows, n_ch):
    rows_per_worker = n_rows // NW
    n_chunks = rows_per_worker // CHUNK

    mesh = plsc.VectorSubcoreMesh(core_axis_name="c", subcore_axis_name="s")

    @functools.partial(
        pl.kernel,
        mesh=mesh,
        out_type=jax.ShapeDtypeStruct((NC, NS, L), jnp.float32),
        compiler_params=pltpu.CompilerParams(needs_layout_passes=False),
        scratch_types=[
            pltpu.VMEM((CHUNK, n_ch), jnp.float32),
            pltpu.VMEM((CHUNK, n_ch), jnp.float32),
            pltpu.VMEM((CHUNK, n_ch), jnp.float32),
            pltpu.VMEM((CHUNK, n_ch), jnp.float32),
            pltpu.VMEM((L,), jnp.float32),
            pltpu.SemaphoreType.DMA,
            pltpu.SemaphoreType.DMA,
            pltpu.SemaphoreType.DMA,
            pltpu.SemaphoreType.DMA,
        ],
    )
    def sc_kernel(o_hbm, b_hbm, out_hbm, o0, o1, b0, b1, stage,
                  so0, so1, sb0, sb1):
        cid = lax.axis_index("c")
        sid = lax.axis_index("s")
        wid = sid * NC + cid
        base_row = wid * rows_per_worker
        obufs = (o0, o1)
        bbufs = (b0, b1)
        osems = (so0, so1)
        bsems = (sb0, sb1)

        def dma_pair(g, par):
            r0 = base_row + g * CHUNK
            oc = pltpu.make_async_copy(
                o_hbm.at[pl.ds(r0, CHUNK), :], obufs[par], osems[par])
            bc = pltpu.make_async_copy(
                b_hbm.at[pl.ds(r0, CHUNK), :], bbufs[par], bsems[par])
            return oc, bc

        def start(g, par):
            oc, bc = dma_pair(g, par)
            oc.start()
            bc.start()

        def wait(g, par):
            oc, bc = dma_pair(g, par)
            oc.wait()
            bc.wait()

        start(0, 0)
        start(1, 1)

        lanes = lax.iota(jnp.int32, L)
        neg_inf = jnp.full((L,), -jnp.inf, jnp.float32)
        zeros = jnp.zeros((L,), jnp.float32)

        def chunk_compute(par, loss):
            for gr in range(CHUNK // L):
                rows = gr * L + lanes

                def body1(t, carry):
                    cols, rb, ro, rm = carry
                    vb = plsc.load_gather(bbufs[par], [rows, cols])
                    vo = plsc.load_gather(obufs[par], [rows, cols])
                    upd = vb > rb
                    return (cols + 1,
                            jnp.where(upd, vb, rb),
                            jnp.where(upd, vo, ro),
                            jnp.maximum(rm, vo))

                cols, rbA, roA, rm = lax.fori_loop(
                    0, n_ch - L, body1, (lanes, neg_inf, zeros, neg_inf),
                    unroll=8)

                def body2(t, carry):
                    cols, wr, rbA, roA, rbB, roB, rm = carry
                    vb = plsc.load_gather(bbufs[par], [rows, cols])
                    vo = plsc.load_gather(obufs[par], [rows, cols])
                    updA = (vb > rbA) & (~wr)
                    updB = (vb > rbB) & wr
                    cols1 = cols + 1
                    wrapnow = cols1 >= n_ch
                    return (jnp.where(wrapnow, cols1 - n_ch, cols1),
                            wr | wrapnow,
                            jnp.where(updA, vb, rbA),
                            jnp.where(updA, vo, roA),
                            jnp.where(updB, vb, rbB),
                            jnp.where(updB, vo, roB),
                            jnp.maximum(rm, vo))

                wr0 = jnp.zeros((L,), jnp.bool_)
                _, _, rbA, roA, rbB, roB, rm = lax.fori_loop(
                    0, L, body2,
                    (cols, wr0, rbA, roA, neg_inf, zeros, rm))

                useB = rbB >= rbA
                ro = jnp.where(useB, roB, roA)
                loss = loss + (1.0 - ro / rm)
            return loss

        def loop_body(i, loss):
            for par in range(2):
                g = 2 * i + par
                wait(g, par)

                @pl.when(g + 2 < n_chunks)
                def _():
                    start(g + 2, par)

                loss = chunk_compute(par, loss)
            return loss

        loss = lax.fori_loop(0, n_chunks // 2, loop_body, zeros)
        stage[...] = loss
        pltpu.sync_copy(stage, out_hbm.at[cid, sid])

    return sc_kernel(o2d, b2d)


TC_BLOCK = 256


def _tc_partial(o2d, b2d, row0, n_rows_tc, n_ch):
    grid = (n_rows_tc // TC_BLOCK,)

    def body(o_ref, b_ref, out_ref):
        @pl.when(pl.program_id(0) == 0)
        def _():
            out_ref[0, 0] = 0.0

        o = o_ref[...]
        b = b_ref[...]
        bmax = jnp.max(b, axis=-1, keepdims=True)
        cols = jax.lax.broadcasted_iota(jnp.int32, (TC_BLOCK, n_ch), 1)
        idx = jnp.min(jnp.where(b == bmax, cols, n_ch), axis=-1,
                      keepdims=True)
        lookup = jnp.max(jnp.where(cols == idx, o, -jnp.inf), axis=-1)
        omax = jnp.max(o, axis=-1)
        out_ref[0, 0] += jnp.sum(1.0 - lookup / omax)

    out = pl.pallas_call(
        body,
        grid=grid,
        in_specs=[
            pl.BlockSpec((TC_BLOCK, n_ch),
                         lambda i: (row0 // TC_BLOCK + i, 0)),
            pl.BlockSpec((TC_BLOCK, n_ch),
                         lambda i: (row0 // TC_BLOCK + i, 0)),
        ],
        out_specs=pl.BlockSpec(memory_space=pltpu.SMEM),
        out_shape=jax.ShapeDtypeStruct((1, 1), jnp.float32),
    )(o2d, b2d)
    return out


def _tc_sum(partials, tc_part):
    def body(x_ref, t_ref, o_ref):
        o_ref[0, 0] = jnp.sum(x_ref[...]) + t_ref[0, 0]

    out = pl.pallas_call(
        body,
        out_shape=jax.ShapeDtypeStruct((1, 1), jnp.float32),
        in_specs=[pl.BlockSpec(memory_space=pltpu.VMEM),
                  pl.BlockSpec(memory_space=pltpu.SMEM)],
        out_specs=pl.BlockSpec(memory_space=pltpu.SMEM),
    )(partials, tc_part)
    return out[0, 0]


SC_ROWS = 81920


def kernel(original, boosted):
    n_ch = original.shape[-1]
    n_rows = original.size // n_ch
    o2d = original.reshape(n_rows, n_ch)
    b2d = boosted.reshape(n_rows, n_ch)
    assert SC_ROWS % (NW * CHUNK) == 0
    assert (n_rows - SC_ROWS) % TC_BLOCK == 0
    partials = _sc_partials(o2d, b2d, SC_ROWS, n_ch)
    tc_part = _tc_partial(o2d, b2d, SC_ROWS, n_rows - SC_ROWS, n_ch)
    return _tc_sum(partials, tc_part)


# speedup vs baseline: 1.3661x; 1.3661x over previous
import functools

import jax
import jax.numpy as jnp
from jax import lax
from jax.experimental import pallas as pl
from jax.experimental.pallas import tpu as pltpu
from jax.experimental.pallas import tpu_sc as plsc

NC = 2
NS = 16
L = 16
NW = NC * NS
CHUNK = 64


def _sc_partials(o2d, b2d, n_rows, n_ch):
    rows_per_worker = n_rows // NW
    n_chunks = rows_per_worker // CHUNK

    mesh = plsc.VectorSubcoreMesh(core_axis_name="c", subcore_axis_name="s")

    @functools.partial(
        pl.kernel,
        mesh=mesh,
        out_type=jax.ShapeDtypeStruct((NC, NS, L), jnp.float32),
        compiler_params=pltpu.CompilerParams(needs_layout_passes=False),
        scratch_types=[
            pltpu.VMEM((CHUNK, n_ch), jnp.float32),
            pltpu.VMEM((CHUNK, n_ch), jnp.float32),
            pltpu.VMEM((CHUNK, n_ch), jnp.float32),
            pltpu.VMEM((CHUNK, n_ch), jnp.float32),
            pltpu.VMEM((L,), jnp.float32),
            pltpu.SemaphoreType.DMA,
            pltpu.SemaphoreType.DMA,
            pltpu.SemaphoreType.DMA,
            pltpu.SemaphoreType.DMA,
        ],
    )
    def sc_kernel(o_hbm, b_hbm, out_hbm, o0, o1, b0, b1, stage,
                  so0, so1, sb0, sb1):
        cid = lax.axis_index("c")
        sid = lax.axis_index("s")
        wid = sid * NC + cid
        base_row = wid * rows_per_worker
        obufs = (o0, o1)
        bbufs = (b0, b1)
        osems = (so0, so1)
        bsems = (sb0, sb1)

        def dma_pair(g, par):
            r0 = base_row + g * CHUNK
            oc = pltpu.make_async_copy(
                o_hbm.at[pl.ds(r0, CHUNK), :], obufs[par], osems[par])
            bc = pltpu.make_async_copy(
                b_hbm.at[pl.ds(r0, CHUNK), :], bbufs[par], bsems[par])
            return oc, bc

        def start(g, par):
            oc, bc = dma_pair(g, par)
            oc.start()
            bc.start()

        def wait(g, par):
            oc, bc = dma_pair(g, par)
            oc.wait()
            bc.wait()

        start(0, 0)
        start(1, 1)

        lanes = lax.iota(jnp.int32, L)
        neg_inf = jnp.full((L,), -jnp.inf, jnp.float32)
        zeros = jnp.zeros((L,), jnp.float32)

        def chunk_compute(par, loss):
            for gr in range(CHUNK // L):
                rows = gr * L + lanes

                def body1(t, carry):
                    cols, rb, ro, rm = carry
                    vb = plsc.load_gather(bbufs[par], [rows, cols])
                    vo = plsc.load_gather(obufs[par], [rows, cols])
                    upd = vb > rb
                    return (cols + 1,
                            jnp.where(upd, vb, rb),
                            jnp.where(upd, vo, ro),
                            jnp.maximum(rm, vo))

                cols, rbA, roA, rm = lax.fori_loop(
                    0, n_ch - L, body1, (lanes, neg_inf, zeros, neg_inf),
                    unroll=8)

                def body2(t, carry):
                    cols, wr, rbA, roA, rbB, roB, rm = carry
                    vb = plsc.load_gather(bbufs[par], [rows, cols])
                    vo = plsc.load_gather(obufs[par], [rows, cols])
                    updA = (vb > rbA) & (~wr)
                    updB = (vb > rbB) & wr
                    cols1 = cols + 1
                    wrapnow = cols1 >= n_ch
                    return (jnp.where(wrapnow, cols1 - n_ch, cols1),
                            wr | wrapnow,
                            jnp.where(updA, vb, rbA),
                            jnp.where(updA, vo, roA),
                            jnp.where(updB, vb, rbB),
                            jnp.where(updB, vo, roB),
                            jnp.maximum(rm, vo))

                wr0 = jnp.zeros((L,), jnp.bool_)
                _, _, rbA, roA, rbB, roB, rm = lax.fori_loop(
                    0, L, body2,
                    (cols, wr0, rbA, roA, neg_inf, zeros, rm))

                useB = rbB >= rbA
                ro = jnp.where(useB, roB, roA)
                loss = loss + (1.0 - ro / rm)
            return loss

        def loop_body(i, loss):
            for par in range(2):
                g = 2 * i + par
                wait(g, par)

                @pl.when(g + 2 < n_chunks)
                def _():
                    start(g + 2, par)

                loss = chunk_compute(par, loss)
            return loss

        loss = lax.fori_loop(0, n_chunks // 2, loop_body, zeros)
        stage[...] = loss
        pltpu.sync_copy(stage, out_hbm.at[cid, sid])

    return sc_kernel(o2d, b2d)


TC_BLOCK = 512


def _tc_partial(o2d, b2d, row0, n_rows_tc, n_ch):
    grid = (n_rows_tc // TC_BLOCK,)

    def body(o_ref, b_ref, out_ref):
        @pl.when(pl.program_id(0) == 0)
        def _():
            out_ref[0, 0] = 0.0

        o = o_ref[...]
        b = b_ref[...]
        bmax = jnp.max(b, axis=-1, keepdims=True)
        cols = jax.lax.broadcasted_iota(jnp.int32, (TC_BLOCK, n_ch), 1)
        idx = jnp.min(jnp.where(b == bmax, cols, n_ch), axis=-1,
                      keepdims=True)
        lookup = jnp.max(jnp.where(cols == idx, o, -jnp.inf), axis=-1)
        omax = jnp.max(o, axis=-1)
        out_ref[0, 0] += jnp.sum(1.0 - lookup / omax)

    out = pl.pallas_call(
        body,
        grid=grid,
        in_specs=[
            pl.BlockSpec((TC_BLOCK, n_ch),
                         lambda i: (row0 // TC_BLOCK + i, 0)),
            pl.BlockSpec((TC_BLOCK, n_ch),
                         lambda i: (row0 // TC_BLOCK + i, 0)),
        ],
        out_specs=pl.BlockSpec(memory_space=pltpu.SMEM),
        out_shape=jax.ShapeDtypeStruct((1, 1), jnp.float32),
    )(o2d, b2d)
    return out


def _tc_sum(partials, tc_part):
    def body(x_ref, t_ref, o_ref):
        o_ref[0, 0] = jnp.sum(x_ref[...]) + t_ref[0, 0]

    out = pl.pallas_call(
        body,
        out_shape=jax.ShapeDtypeStruct((1, 1), jnp.float32),
        in_specs=[pl.BlockSpec(memory_space=pltpu.VMEM),
                  pl.BlockSpec(memory_space=pltpu.SMEM)],
        out_specs=pl.BlockSpec(memory_space=pltpu.SMEM),
    )(partials, tc_part)
    return out[0, 0]


SC_ROWS = 86016


def kernel(original, boosted):
    n_ch = original.shape[-1]
    n_rows = original.size // n_ch
    o2d = original.reshape(n_rows, n_ch)
    b2d = boosted.reshape(n_rows, n_ch)
    assert SC_ROWS % (NW * CHUNK) == 0
    assert (n_rows - SC_ROWS) % TC_BLOCK == 0
    partials = _sc_partials(o2d, b2d, SC_ROWS, n_ch)
    tc_part = _tc_partial(o2d, b2d, SC_ROWS, n_rows - SC_ROWS, n_ch)
    return _tc_sum(partials, tc_part)
